# baseline (speedup 1.0000x reference)
import functools

import jax
import jax.numpy as jnp
from jax import lax
from jax.experimental import pallas as pl
from jax.experimental.pallas import tpu as pltpu
from jax.experimental.pallas import tpu_sc as plsc

_N = 50000
_H = 16
_E = 800000
_NW = 32
_EW = _E // _NW
_MC = 1000
_NMC = _EW // _MC
_NR = 50048
_SPW = _NR // 16
_BN = 2000
_GRID = _N // _BN


def _make_sc_pass(first):
    mesh = plsc.VectorSubcoreMesh(core_axis_name="c", subcore_axis_name="s")
    out_type = jax.ShapeDtypeStruct((2, _NR, _H), jnp.float32)
    mc, nmc = _MC, _NMC
    scratch = [
        pltpu.VMEM((2, mc), jnp.int32),
        pltpu.VMEM((2, mc), jnp.int32),
        pltpu.VMEM((2, mc, _H), jnp.float32),
        pltpu.VMEM_SHARED((_NR, _H), jnp.float32),
        pltpu.SemaphoreType.DMA((2,)),
    ]
    if first:
        scratch = scratch[:3] + [
            pltpu.VMEM((2, mc), jnp.int32),
            pltpu.VMEM((2, mc, _H), jnp.float32),
            pltpu.SemaphoreType.DMA((2,)),
        ] + scratch[3:]

    def body_fn(table, srcp, dstp, zrows, *rest):
        if first:
            sidx, tab3, out, src_v, dst_v, rows_v, sidx_v, ev_v, sem_e, acc, sem = rest
        else:
            out, src_v, dst_v, rows_v, acc, sem = rest
        c = lax.axis_index("c")
        s = lax.axis_index("s")
        wid = s * 2 + c
        pltpu.sync_copy(zrows, acc.at[pl.ds(s * _SPW, _SPW), :])
        plsc.subcore_barrier()
        base = wid * _EW

        def fetch(j, p):
            pltpu.sync_copy(srcp.at[pl.ds(base + j * mc, mc)], src_v.at[p])
            pltpu.sync_copy(dstp.at[pl.ds(base + j * mc, mc)], dst_v.at[p])
            pltpu.async_copy(table.at[src_v.at[p]], rows_v.at[p], sem.at[p])
            if first:
                pltpu.sync_copy(sidx.at[pl.ds(base + j * mc, mc)],
                                sidx_v.at[p])
                pltpu.async_copy(tab3.at[sidx_v.at[p]], ev_v.at[p],
                                 sem_e.at[p])

        def consume(p):
            pltpu.make_async_copy(
                table.at[src_v.at[p]], rows_v.at[p], sem.at[p]).wait()
            pltpu.sync_copy(rows_v.at[p], acc.at[dst_v.at[p]], add=True)
            if first:
                pltpu.make_async_copy(
                    tab3.at[sidx_v.at[p]], ev_v.at[p], sem_e.at[p]).wait()
                pltpu.sync_copy(ev_v.at[p], acc.at[dst_v.at[p]], add=True)

        fetch(0, 0)

        def body(t, carry):
            for p in (0, 1):
                tc = 2 * t + p

                @pl.when(tc + 1 < nmc)
                def _():
                    fetch(tc + 1, 1 - p)

                consume(p)
            return carry

        lax.fori_loop(0, nmc // 2, body, 0)
        if nmc % 2:
            consume(0)
        plsc.subcore_barrier()
        pltpu.sync_copy(acc.at[pl.ds(s * _SPW, _SPW), :],
                        out.at[c, pl.ds(s * _SPW, _SPW), :])

    return pl.kernel(
        body_fn,
        mesh=mesh,
        out_type=out_type,
        scratch_types=scratch,
        compiler_params=pltpu.CompilerParams(use_tc_tiling_on_sc=False),
    )


_sc_pass_first = _make_sc_pass(True)
_sc_pass = _make_sc_pass(False)


def _l1_body(P_ref, x_ref, wd_ref, wb_ref, wc_ref, b_ref,
             x2_ref, m_ref, ic_ref, gm_ref):
    P = P_ref[0] + P_ref[1]
    s1 = P[:, 0:1]
    cnt = P[:, 1:2]
    g = P[:, 2:3]
    ic = 1.0 / jnp.maximum(cnt, 1.0)
    m = (cnt > 0.0).astype(jnp.float32)
    gm = g * ic
    xb = x_ref[...]
    x2_ref[...] = (m * (xb * wd_ref[...] + b_ref[...])
                   + (ic * s1) * wb_ref[...] + gm * wc_ref[...])
    m_ref[...] = m
    ic_ref[...] = ic
    gm_ref[...] = gm


def _tc_layer1(P, x, W_in, b_in):
    wd = W_in[0:1] - W_in[1:2]
    wb = W_in[1:2]
    wc = W_in[2:3]
    b = b_in.reshape(1, _H)
    vspec = pl.BlockSpec((_BN, 1), lambda i: (i, 0))
    wspec = pl.BlockSpec((1, _H), lambda i: (0, 0))
    return pl.pallas_call(
        _l1_body,
        grid=(_GRID,),
        in_specs=[
            pl.BlockSpec((2, _BN, _H), lambda i: (0, i, 0)),
            vspec, wspec, wspec, wspec, wspec,
        ],
        out_specs=[
            pl.BlockSpec((_BN, _H), lambda i: (i, 0)),
            vspec, vspec, vspec,
        ],
        out_shape=[
            jax.ShapeDtypeStruct((_N, _H), jnp.float32),
            jax.ShapeDtypeStruct((_N, 1), jnp.float32),
            jax.ShapeDtypeStruct((_N, 1), jnp.float32),
            jax.ShapeDtypeStruct((_N, 1), jnp.float32),
        ],
    )(P, x, wd, wb, wc, b)


def _make_layer_body(act, has_res):
    def body(h_ref, P_ref, m_ref, ic_ref, gm_ref, *rest):
        if has_res:
            res_ref = rest[0]
            rest = rest[1:]
        wd_ref, wb_ref, wc_ref, b_ref, o_ref = rest
        P = P_ref[0] + P_ref[1]
        o = (m_ref[...] * (jnp.dot(h_ref[...], wd_ref[...],
                                   preferred_element_type=jnp.float32)
                           + b_ref[...])
             + jnp.dot(ic_ref[...] * P, wb_ref[...],
                       preferred_element_type=jnp.float32)
             + gm_ref[...] * wc_ref[...])
        if has_res:
            o = o + res_ref[...]
        if act:
            o = jnp.where(o >= 0, o, 0.01 * o)
        o_ref[...] = o
    return body


def _tc_layer(h, P, m, ic, gm, W, b, res, act):
    wd = W[:_H] - W[_H:2 * _H]
    wb = W[_H:2 * _H]
    wc = W[2 * _H:2 * _H + 1]
    ho = W.shape[1]
    bb = b.reshape(1, ho)
    vspec = pl.BlockSpec((_BN, 1), lambda i: (i, 0))
    wspec = pl.BlockSpec((_H, ho), lambda i: (0, 0))
    sspec = pl.BlockSpec((1, ho), lambda i: (0, 0))
    ins = [h, P, m, ic, gm]
    specs = [
        pl.BlockSpec((_BN, _H), lambda i: (i, 0)),
        pl.BlockSpec((2, _BN, _H), lambda i: (0, i, 0)),
        vspec, vspec, vspec,
    ]
    if res is not None:
        ins.append(res)
        rc = res.shape[1]
        specs.append(pl.BlockSpec((_BN, rc), lambda i: (i, 0)))
    ins += [wd, wb, wc, bb]
    specs += [wspec, wspec, sspec, sspec]
    return pl.pallas_call(
        _make_layer_body(act, res is not None),
        grid=(_GRID,),
        in_specs=specs,
        out_specs=pl.BlockSpec((_BN, ho), lambda i: (i, 0)),
        out_shape=jax.ShapeDtypeStruct((_N, ho), jnp.float32),
    )(*ins)


def kernel(x, edge_index, W_in, b_in, W1, b1, W2, b2, W3, b3, W4, b4,
           W5, b5, W6, b6, W_out, b_out):
    srcp = edge_index[0].astype(jnp.int32)
    dstp = edge_index[1].astype(jnp.int32)
    zrows = jnp.zeros((_SPW, _H), jnp.float32)
    sidx = (jnp.sign(srcp - dstp) + 1).astype(jnp.int32)
    tab3 = jnp.zeros((8, _H), jnp.float32).at[0, 2].set(-1.0).at[2, 2].set(1.0)
    T0 = jnp.concatenate(
        [x, jnp.ones((_N, 1), jnp.float32),
         jnp.zeros((_N, _H - 2), jnp.float32)], axis=1)

    P1 = _sc_pass_first(T0, srcp, dstp, zrows, sidx, tab3)
    x2, m, ic, gm = _tc_layer1(P1, x, W_in, b_in)

    Ws = [(W1, b1), (W2, b2), (W3, b3), (W4, b4), (W5, b5), (W6, b6)]
    for i in range(0, 6, 2):
        P = _sc_pass(x2, srcp, dstp, zrows)
        x1 = _tc_layer(x2, P, m, ic, gm, *Ws[i], res=None, act=True)
        P = _sc_pass(x1, srcp, dstp, zrows)
        x2 = _tc_layer(x1, P, m, ic, gm, *Ws[i + 1], res=x2, act=True)

    P = _sc_pass(x2, srcp, dstp, zrows)
    Wop = jnp.pad(W_out, ((0, 0), (0, _H - 1)))
    bop = jnp.pad(b_out, ((0, _H - 1),))
    y = _tc_layer(x2, P, m, ic, gm, Wop, bop, res=x, act=False)
    return y[:, 0:1]


# speedup vs baseline: 3.2171x; 3.2171x over previous
import functools

import jax
import jax.numpy as jnp
from jax import lax
from jax.experimental import pallas as pl
from jax.experimental.pallas import tpu as pltpu
from jax.experimental.pallas import tpu_sc as plsc

_N = 50000
_H = 16
_E = 800000
_NW = 32
_EW = _E // _NW
_MC = 1000
_NMC = _EW // _MC
_NR = 50048
_SPW = _NR // 16
_BN = 2000
_GRID = _N // _BN


def _make_sc_pass(nrows):
    mesh = plsc.VectorSubcoreMesh(core_axis_name="c", subcore_axis_name="s")
    out_type = jax.ShapeDtypeStruct((2, _NR, _H), jnp.float32)
    mc, nmc = _MC, _NMC
    scratch = [
        pltpu.VMEM((2, mc), jnp.int32),
        pltpu.VMEM((2, mc), jnp.int32),
        pltpu.VMEM((2, mc, _H), jnp.float32),
        pltpu.VMEM_SHARED((_NR, _H), jnp.float32),
        pltpu.SemaphoreType.DMA((2,)),
    ]
    del nrows

    def body_fn(table, srcp, dstp, zrows, out, src_v, dst_v, rows_v, acc, sem):
        c = lax.axis_index("c")
        s = lax.axis_index("s")
        wid = s * 2 + c
        pltpu.sync_copy(zrows, acc.at[pl.ds(s * _SPW, _SPW), :])
        plsc.subcore_barrier()
        base = wid * _EW

        def fetch(j, p):
            pltpu.sync_copy(srcp.at[pl.ds(base + j * mc, mc)], src_v.at[p])
            pltpu.sync_copy(dstp.at[pl.ds(base + j * mc, mc)], dst_v.at[p])
            pltpu.async_copy(table.at[src_v.at[p]], rows_v.at[p], sem.at[p])

        def consume(p):
            pltpu.make_async_copy(
                table.at[src_v.at[p]], rows_v.at[p], sem.at[p]).wait()
            pltpu.sync_copy(rows_v.at[p], acc.at[dst_v.at[p]], add=True)

        fetch(0, 0)

        def body(t, carry):
            for p in (0, 1):
                tc = 2 * t + p

                @pl.when(tc + 1 < nmc)
                def _():
                    fetch(tc + 1, 1 - p)

                consume(p)
            return carry

        lax.fori_loop(0, nmc // 2, body, 0)
        if nmc % 2:
            consume(0)
        plsc.subcore_barrier()
        pltpu.sync_copy(acc.at[pl.ds(s * _SPW, _SPW), :],
                        out.at[c, pl.ds(s * _SPW, _SPW), :])

    return pl.kernel(
        body_fn,
        mesh=mesh,
        out_type=out_type,
        scratch_types=scratch,
        compiler_params=pltpu.CompilerParams(use_tc_tiling_on_sc=False),
    )


_sc_pass_first = _make_sc_pass(3 * _N)
_sc_pass = _make_sc_pass(_N)


def _l1_body(P_ref, x_ref, wd_ref, wb_ref, wc_ref, b_ref,
             x2_ref, m_ref, ic_ref, gm_ref):
    P = P_ref[0] + P_ref[1]
    s1 = P[:, 0:1]
    cnt = P[:, 1:2]
    g = P[:, 2:3]
    ic = 1.0 / jnp.maximum(cnt, 1.0)
    m = (cnt > 0.0).astype(jnp.float32)
    gm = g * ic
    xb = x_ref[...]
    x2_ref[...] = (m * (xb * wd_ref[...] + b_ref[...])
                   + (ic * s1) * wb_ref[...] + gm * wc_ref[...])
    m_ref[...] = m
    ic_ref[...] = ic
    gm_ref[...] = gm


def _tc_layer1(P, x, W_in, b_in):
    wd = W_in[0:1] - W_in[1:2]
    wb = W_in[1:2]
    wc = W_in[2:3]
    b = b_in.reshape(1, _H)
    vspec = pl.BlockSpec((_BN, 1), lambda i: (i, 0))
    wspec = pl.BlockSpec((1, _H), lambda i: (0, 0))
    return pl.pallas_call(
        _l1_body,
        grid=(_GRID,),
        in_specs=[
            pl.BlockSpec((2, _BN, _H), lambda i: (0, i, 0)),
            vspec, wspec, wspec, wspec, wspec,
        ],
        out_specs=[
            pl.BlockSpec((_BN, _H), lambda i: (i, 0)),
            vspec, vspec, vspec,
        ],
        out_shape=[
            jax.ShapeDtypeStruct((_N, _H), jnp.float32),
            jax.ShapeDtypeStruct((_N, 1), jnp.float32),
            jax.ShapeDtypeStruct((_N, 1), jnp.float32),
            jax.ShapeDtypeStruct((_N, 1), jnp.float32),
        ],
    )(P, x, wd, wb, wc, b)


def _make_layer_body(act, has_res):
    def body(h_ref, P_ref, m_ref, ic_ref, gm_ref, *rest):
        if has_res:
            res_ref = rest[0]
            rest = rest[1:]
        wd_ref, wb_ref, wc_ref, b_ref, o_ref = rest
        P = P_ref[0] + P_ref[1]
        o = (m_ref[...] * (jnp.dot(h_ref[...], wd_ref[...],
                                   preferred_element_type=jnp.float32)
                           + b_ref[...])
             + jnp.dot(ic_ref[...] * P, wb_ref[...],
                       preferred_element_type=jnp.float32)
             + gm_ref[...] * wc_ref[...])
        if has_res:
            o = o + res_ref[...]
        if act:
            o = jnp.where(o >= 0, o, 0.01 * o)
        o_ref[...] = o
    return body


def _tc_layer(h, P, m, ic, gm, W, b, res, act):
    wd = W[:_H] - W[_H:2 * _H]
    wb = W[_H:2 * _H]
    wc = W[2 * _H:2 * _H + 1]
    ho = W.shape[1]
    bb = b.reshape(1, ho)
    vspec = pl.BlockSpec((_BN, 1), lambda i: (i, 0))
    wspec = pl.BlockSpec((_H, ho), lambda i: (0, 0))
    sspec = pl.BlockSpec((1, ho), lambda i: (0, 0))
    ins = [h, P, m, ic, gm]
    specs = [
        pl.BlockSpec((_BN, _H), lambda i: (i, 0)),
        pl.BlockSpec((2, _BN, _H), lambda i: (0, i, 0)),
        vspec, vspec, vspec,
    ]
    if res is not None:
        ins.append(res)
        rc = res.shape[1]
        specs.append(pl.BlockSpec((_BN, rc), lambda i: (i, 0)))
    ins += [wd, wb, wc, bb]
    specs += [wspec, wspec, sspec, sspec]
    return pl.pallas_call(
        _make_layer_body(act, res is not None),
        grid=(_GRID,),
        in_specs=specs,
        out_specs=pl.BlockSpec((_BN, ho), lambda i: (i, 0)),
        out_shape=jax.ShapeDtypeStruct((_N, ho), jnp.float32),
    )(*ins)


def kernel(x, edge_index, W_in, b_in, W1, b1, W2, b2, W3, b3, W4, b4,
           W5, b5, W6, b6, W_out, b_out):
    srcp = edge_index[0].astype(jnp.int32)
    dstp = edge_index[1].astype(jnp.int32)
    zrows = jnp.zeros((_SPW, _H), jnp.float32)
    sidx2 = 3 * srcp + (jnp.sign(srcp - dstp) + 1).astype(jnp.int32)
    base16 = jnp.concatenate(
        [x, jnp.ones((_N, 1), jnp.float32),
         jnp.zeros((_N, _H - 2), jnp.float32)], axis=1)
    e2 = jnp.zeros((3, _H), jnp.float32).at[0, 2].set(-1.0).at[2, 2].set(1.0)
    T0x = (base16[:, None, :] + e2[None, :, :]).reshape(3 * _N, _H)

    P1 = _sc_pass_first(T0x, sidx2, dstp, zrows)
    x2, m, ic, gm = _tc_layer1(P1, x, W_in, b_in)

    Ws = [(W1, b1), (W2, b2), (W3, b3), (W4, b4), (W5, b5), (W6, b6)]
    for i in range(0, 6, 2):
        P = _sc_pass(x2, srcp, dstp, zrows)
        x1 = _tc_layer(x2, P, m, ic, gm, *Ws[i], res=None, act=True)
        P = _sc_pass(x1, srcp, dstp, zrows)
        x2 = _tc_layer(x1, P, m, ic, gm, *Ws[i + 1], res=x2, act=True)

    P = _sc_pass(x2, srcp, dstp, zrows)
    Wop = jnp.pad(W_out, ((0, 0), (0, _H - 1)))
    bop = jnp.pad(b_out, ((0, _H - 1),))
    y = _tc_layer(x2, P, m, ic, gm, Wop, bop, res=x, act=False)
    return y[:, 0:1]


# speedup vs baseline: 6.0194x; 1.8710x over previous
import functools

import jax
import jax.numpy as jnp
from jax import lax
from jax.experimental import pallas as pl
from jax.experimental.pallas import tpu as pltpu
from jax.experimental.pallas import tpu_sc as plsc

_N = 50000
_H = 16
_E = 800000
_NW = 32
_EW = _E // _NW
_MC = 1000
_NMC = _EW // _MC
_NR = 50048
_SPW = _NR // 16


def _make_sc_pass(nrows):
    mesh = plsc.VectorSubcoreMesh(core_axis_name="c", subcore_axis_name="s")
    out_type = jax.ShapeDtypeStruct((2, _NR, _H), jnp.float32)
    mc, nmc = _MC, _NMC
    scratch = [
        pltpu.VMEM((2, mc), jnp.int32),
        pltpu.VMEM((2, mc), jnp.int32),
        pltpu.VMEM((2, mc, _H), jnp.float32),
        pltpu.VMEM_SHARED((_NR, _H), jnp.float32),
        pltpu.SemaphoreType.DMA((2,)),
    ]
    del nrows

    def body_fn(table, srcp, dstp, zrows, out, src_v, dst_v, rows_v, acc, sem):
        c = lax.axis_index("c")
        s = lax.axis_index("s")
        wid = s * 2 + c
        pltpu.sync_copy(zrows, acc.at[pl.ds(s * _SPW, _SPW), :])
        plsc.subcore_barrier()
        base = wid * _EW

        def fetch(j, p):
            pltpu.sync_copy(srcp.at[pl.ds(base + j * mc, mc)], src_v.at[p])
            pltpu.sync_copy(dstp.at[pl.ds(base + j * mc, mc)], dst_v.at[p])
            pltpu.async_copy(table.at[src_v.at[p]], rows_v.at[p], sem.at[p])

        def consume(p):
            pltpu.make_async_copy(
                table.at[src_v.at[p]], rows_v.at[p], sem.at[p]).wait()
            pltpu.sync_copy(rows_v.at[p], acc.at[dst_v.at[p]], add=True)

        fetch(0, 0)

        def body(t, carry):
            for p in (0, 1):
                tc = 2 * t + p

                @pl.when(tc + 1 < nmc)
                def _():
                    fetch(tc + 1, 1 - p)

                consume(p)
            return carry

        lax.fori_loop(0, nmc // 2, body, 0)
        if nmc % 2:
            consume(0)
        plsc.subcore_barrier()
        pltpu.sync_copy(acc.at[pl.ds(s * _SPW, _SPW), :],
                        out.at[c, pl.ds(s * _SPW, _SPW), :])

    return pl.kernel(
        body_fn,
        mesh=mesh,
        out_type=out_type,
        scratch_types=scratch,
        compiler_params=pltpu.CompilerParams(use_tc_tiling_on_sc=False),
    )


_sc_pass_first = _make_sc_pass(3 * _N)
_sc_pass = _make_sc_pass(_N)


_PF = _NR * _H // 128
_FL = _PF
_BR = _PF
_GRID = 1


def _l1_body(P_ref, xf_ref, wdT_ref, wbT_ref, wcT_ref, bT_ref,
             B0_ref, B1_ref, B2_ref, x2_ref, m_ref, ic_ref, gm_ref):
    Pb = P_ref[0] + P_ref[1]
    f32 = jnp.float32
    sB = jnp.dot(Pb, B0_ref[...], preferred_element_type=f32)
    cB = jnp.dot(Pb, B1_ref[...], preferred_element_type=f32)
    gB = jnp.dot(Pb, B2_ref[...], preferred_element_type=f32)
    ic = 1.0 / jnp.maximum(cB, 1.0)
    m = (cB > 0.0).astype(f32)
    gm = gB * ic
    x2_ref[...] = (m * (xf_ref[...] * wdT_ref[...] + bT_ref[...])
                   + (ic * sB) * wbT_ref[...] + gm * wcT_ref[...])
    m_ref[...] = m
    ic_ref[...] = ic
    gm_ref[...] = gm


def _tc_layer1(P, xf, W_in, b_in):
    wd = W_in[0] - W_in[1]
    ey = jnp.eye(8, dtype=jnp.float32)
    tile = lambda v: jnp.tile(v, 8)[None, :]
    Bs = []
    for k in range(3):
        Mk = jnp.zeros((_H, _H), jnp.float32).at[k].set(1.0)
        Bs.append(jnp.kron(ey, Mk))
    fspec = pl.BlockSpec((_BR, 128), lambda i: (i, 0))
    wspec = pl.BlockSpec((128, 128), lambda i: (0, 0))
    sspec = pl.BlockSpec((1, 128), lambda i: (0, 0))
    fshape = jax.ShapeDtypeStruct((_FL, 128), jnp.float32)
    return pl.pallas_call(
        _l1_body,
        grid=(_GRID,),
        in_specs=[pl.BlockSpec((2, _BR, 128), lambda i: (0, i, 0)),
                  fspec, sspec, sspec, sspec, sspec, wspec, wspec, wspec],
        out_specs=[fspec, fspec, fspec, fspec],
        out_shape=[fshape, fshape, fshape, fshape],
    )(P, xf, tile(wd), tile(W_in[1]), tile(W_in[2]), tile(b_in), *Bs)


def _make_layer_body(act, has_res):
    def body(h_ref, P_ref, m_ref, ic_ref, gm_ref, *rest):
        if has_res:
            res_ref = rest[0]
            rest = rest[1:]
        wd_ref, wb_ref, wc_ref, b_ref, o_ref = rest
        P = P_ref[0] + P_ref[1]
        o = (m_ref[...] * (jnp.dot(h_ref[...], wd_ref[...],
                                   preferred_element_type=jnp.float32)
                           + b_ref[...])
             + jnp.dot(ic_ref[...] * P, wb_ref[...],
                       preferred_element_type=jnp.float32)
             + gm_ref[...] * wc_ref[...])
        if has_res:
            o = o + res_ref[...]
        if act:
            o = jnp.where(o >= 0, o, 0.01 * o)
        o_ref[...] = o
    return body


def _tc_layer(h, P, m, ic, gm, WdBD, WbBD, wcT, bT, res, act):
    fspec = pl.BlockSpec((_BR, 128), lambda i: (i, 0))
    wspec = pl.BlockSpec((128, 128), lambda i: (0, 0))
    sspec = pl.BlockSpec((1, 128), lambda i: (0, 0))
    ins = [h, P, m, ic, gm]
    specs = [fspec, pl.BlockSpec((2, _BR, 128), lambda i: (0, i, 0)),
             fspec, fspec, fspec]
    if res is not None:
        ins.append(res)
        specs.append(fspec)
    ins += [WdBD, WbBD, wcT, bT]
    specs += [wspec, wspec, sspec, sspec]
    return pl.pallas_call(
        _make_layer_body(act, res is not None),
        grid=(_GRID,),
        in_specs=specs,
        out_specs=fspec,
        out_shape=jax.ShapeDtypeStruct((_FL, 128), jnp.float32),
    )(*ins)


def _prep_w(W, b):
    ey = jnp.eye(8, dtype=jnp.float32)
    wd = W[:_H] - W[_H:2 * _H]
    wb = W[_H:2 * _H]
    if W.shape[1] == 1:
        on = jnp.ones((1, _H), jnp.float32)
        wd, wb = wd @ on, wb @ on
        wc = jnp.tile(W[2 * _H] @ on, 8)[None, :]
        bT = jnp.tile(b @ on, 8).reshape(1, 128)
    else:
        wc = jnp.tile(W[2 * _H], 8)[None, :]
        bT = jnp.tile(b, 8)[None, :]
    return jnp.kron(ey, wd), jnp.kron(ey, wb), wc, bT


def kernel(x, edge_index, W_in, b_in, W1, b1, W2, b2, W3, b3, W4, b4,
           W5, b5, W6, b6, W_out, b_out):
    srcp = edge_index[0].astype(jnp.int32)
    dstp = edge_index[1].astype(jnp.int32)
    zrows = jnp.zeros((_SPW, _H), jnp.float32)
    sidx2 = 3 * srcp + (jnp.sign(srcp - dstp) + 1).astype(jnp.int32)
    base16 = jnp.concatenate(
        [x, jnp.ones((_N, 1), jnp.float32),
         jnp.zeros((_N, _H - 2), jnp.float32)], axis=1)
    e2 = jnp.zeros((3, _H), jnp.float32).at[0, 2].set(-1.0).at[2, 2].set(1.0)
    T0x = (base16[:, None, :] + e2[None, :, :]).reshape(3 * _N, _H)

    xf = jnp.pad(jnp.repeat(x[:, 0], _H),
                 (0, (_NR - _N) * _H)).reshape(_FL, 128)
    P1 = _sc_pass_first(T0x, sidx2, dstp, zrows).reshape(2, _PF, 128)
    x2f, m, ic, gm = _tc_layer1(P1, xf, W_in, b_in)

    Ws = [(W1, b1), (W2, b2), (W3, b3), (W4, b4), (W5, b5), (W6, b6)]
    for i in range(0, 6, 2):
        P = _sc_pass(x2f.reshape(_NR, _H), srcp, dstp, zrows).reshape(2, _PF, 128)
        x1f = _tc_layer(x2f, P, m, ic, gm, *_prep_w(*Ws[i]), res=None, act=True)
        P = _sc_pass(x1f.reshape(_NR, _H), srcp, dstp, zrows).reshape(2, _PF, 128)
        x2f = _tc_layer(x1f, P, m, ic, gm, *_prep_w(*Ws[i + 1]), res=x2f, act=True)

    P = _sc_pass(x2f.reshape(_NR, _H), srcp, dstp, zrows).reshape(2, _PF, 128)
    yf = _tc_layer(x2f, P, m, ic, gm, *_prep_w(W_out, b_out), res=xf, act=False)
    return yf.reshape(_NR, _H)[:_N, 0:1]


# speedup vs baseline: 6.3321x; 1.0520x over previous
import functools

import jax
import jax.numpy as jnp
from jax import lax
from jax.experimental import pallas as pl
from jax.experimental.pallas import tpu as pltpu
from jax.experimental.pallas import tpu_sc as plsc

_N = 50000
_H = 16
_E = 800000
_NW = 32
_EW = _E // _NW
_MC = 1000
_NMC = _EW // _MC
_NR = 50048
_SPW = _NR // 16


def _make_sc_pass(nrows):
    mesh = plsc.VectorSubcoreMesh(core_axis_name="c", subcore_axis_name="s")
    out_type = jax.ShapeDtypeStruct((2, _NR, _H), jnp.float32)
    mc, nmc = _MC, _NMC
    scratch = [
        pltpu.VMEM((2, mc), jnp.int32),
        pltpu.VMEM((2, mc), jnp.int32),
        pltpu.VMEM((2, mc, _H), jnp.float32),
        pltpu.VMEM_SHARED((_NR, _H), jnp.float32),
        pltpu.SemaphoreType.DMA((2,)),
    ]
    del nrows

    def body_fn(table, srcp, dstp, zrows, out, src_v, dst_v, rows_v, acc, sem):
        c = lax.axis_index("c")
        s = lax.axis_index("s")
        wid = s * 2 + c
        pltpu.sync_copy(zrows, acc.at[pl.ds(s * _SPW, _SPW), :])
        plsc.subcore_barrier()
        base = wid * _EW

        def fetch(j, p):
            pltpu.sync_copy(srcp.at[pl.ds(base + j * mc, mc)], src_v.at[p])
            pltpu.sync_copy(dstp.at[pl.ds(base + j * mc, mc)], dst_v.at[p])
            pltpu.async_copy(table.at[src_v.at[p]], rows_v.at[p], sem.at[p])

        def consume(p):
            pltpu.make_async_copy(
                table.at[src_v.at[p]], rows_v.at[p], sem.at[p]).wait()
            pltpu.sync_copy(rows_v.at[p], acc.at[dst_v.at[p]], add=True)

        fetch(0, 0)

        def body(t, carry):
            for p in (0, 1):
                tc = 2 * t + p

                @pl.when(tc + 1 < nmc)
                def _():
                    fetch(tc + 1, 1 - p)

                consume(p)
            return carry

        lax.fori_loop(0, nmc // 2, body, 0)
        if nmc % 2:
            consume(0)
        plsc.subcore_barrier()
        pltpu.sync_copy(acc.at[pl.ds(s * _SPW, _SPW), :],
                        out.at[c, pl.ds(s * _SPW, _SPW), :])

    return pl.kernel(
        body_fn,
        mesh=mesh,
        out_type=out_type,
        scratch_types=scratch,
        compiler_params=pltpu.CompilerParams(use_tc_tiling_on_sc=False),
    )


_sc_pass_first = _make_sc_pass(3 * _N)
_sc_pass = _make_sc_pass(_N)


_PF = _NR * _H // 128
_FL = _PF
_BR = _PF
_GRID = 1


def _l1_body(P_ref, xf_ref, wdT_ref, wbT_ref, wcT_ref, bT_ref,
             B0_ref, B1_ref, B2_ref, x2_ref, m_ref, ic_ref, gm_ref):
    Pb = P_ref[0] + P_ref[1]
    f32 = jnp.float32
    sB = jnp.dot(Pb, B0_ref[...], preferred_element_type=f32)
    cB = jnp.dot(Pb, B1_ref[...], preferred_element_type=f32)
    gB = jnp.dot(Pb, B2_ref[...], preferred_element_type=f32)
    ic = 1.0 / jnp.maximum(cB, 1.0)
    m = (cB > 0.0).astype(f32)
    gm = gB * ic
    x2_ref[...] = (m * (xf_ref[...] * wdT_ref[...] + bT_ref[...])
                   + (ic * sB) * wbT_ref[...] + gm * wcT_ref[...])
    m_ref[...] = m
    ic_ref[...] = ic
    gm_ref[...] = gm


def _tc_layer1(P, xf, W_in, b_in):
    wd = W_in[0] - W_in[1]
    ey = jnp.eye(8, dtype=jnp.float32)
    tile = lambda v: jnp.tile(v, 8)[None, :]
    Bs = []
    for k in range(3):
        Mk = jnp.zeros((_H, _H), jnp.float32).at[k].set(1.0)
        Bs.append(jnp.kron(ey, Mk))
    fspec = pl.BlockSpec((_BR, 128), lambda i: (i, 0))
    wspec = pl.BlockSpec((128, 128), lambda i: (0, 0))
    sspec = pl.BlockSpec((1, 128), lambda i: (0, 0))
    fshape = jax.ShapeDtypeStruct((_FL, 128), jnp.float32)
    return pl.pallas_call(
        _l1_body,
        grid=(_GRID,),
        in_specs=[pl.BlockSpec((2, _BR, 128), lambda i: (0, i, 0)),
                  fspec, sspec, sspec, sspec, sspec, wspec, wspec, wspec],
        out_specs=[fspec, fspec, fspec, fspec],
        out_shape=[fshape, fshape, fshape, fshape],
    )(P, xf, tile(wd), tile(W_in[1]), tile(W_in[2]), tile(b_in), *Bs)


def _make_layer_body(act, has_res, compact):
    def body(h_ref, P_ref, m_ref, ic_ref, gm_ref, *rest):
        if has_res:
            res_ref = rest[0]
            rest = rest[1:]
        if compact:
            cm_ref = rest[0]
            rest = rest[1:]
        wd_ref, wb_ref, wc_ref, b_ref, o_ref = rest
        P = P_ref[0] + P_ref[1]
        o = (m_ref[...] * (jnp.dot(h_ref[...], wd_ref[...],
                                   preferred_element_type=jnp.float32)
                           + b_ref[...])
             + jnp.dot(ic_ref[...] * P, wb_ref[...],
                       preferred_element_type=jnp.float32)
             + gm_ref[...] * wc_ref[...])
        if has_res:
            o = o + res_ref[...]
        if act:
            o = jnp.where(o >= 0, o, 0.01 * o)
        if compact:
            o = jnp.dot(o, cm_ref[...], preferred_element_type=jnp.float32)
        o_ref[...] = o
    return body


def _tc_layer(h, P, m, ic, gm, WdBD, WbBD, wcT, bT, res, act,
              compact=False):
    fspec = pl.BlockSpec((_BR, 128), lambda i: (i, 0))
    wspec = pl.BlockSpec((128, 128), lambda i: (0, 0))
    sspec = pl.BlockSpec((1, 128), lambda i: (0, 0))
    ins = [h, P, m, ic, gm]
    specs = [fspec, pl.BlockSpec((2, _BR, 128), lambda i: (0, i, 0)),
             fspec, fspec, fspec]
    if res is not None:
        ins.append(res)
        specs.append(fspec)
    if compact:
        cmat = jnp.kron(jnp.eye(8, dtype=jnp.float32),
                        jnp.zeros((_H, 1), jnp.float32).at[0, 0].set(1.0))
        ins.append(cmat)
        specs.append(pl.BlockSpec((128, 8), lambda i: (0, 0)))
        out_spec = pl.BlockSpec((_BR, 8), lambda i: (i, 0))
        out_shape = jax.ShapeDtypeStruct((_FL, 8), jnp.float32)
    else:
        out_spec = fspec
        out_shape = jax.ShapeDtypeStruct((_FL, 128), jnp.float32)
    ins += [WdBD, WbBD, wcT, bT]
    specs += [wspec, wspec, sspec, sspec]
    return pl.pallas_call(
        _make_layer_body(act, res is not None, compact),
        grid=(_GRID,),
        in_specs=specs,
        out_specs=out_spec,
        out_shape=out_shape,
    )(*ins)


def _prep_w(W, b):
    ey = jnp.eye(8, dtype=jnp.float32)
    wd = W[:_H] - W[_H:2 * _H]
    wb = W[_H:2 * _H]
    if W.shape[1] == 1:
        on = jnp.ones((1, _H), jnp.float32)
        wd, wb = wd @ on, wb @ on
        wc = jnp.tile(W[2 * _H] @ on, 8)[None, :]
        bT = jnp.tile(b @ on, 8).reshape(1, 128)
    else:
        wc = jnp.tile(W[2 * _H], 8)[None, :]
        bT = jnp.tile(b, 8)[None, :]
    return jnp.kron(ey, wd), jnp.kron(ey, wb), wc, bT


def kernel(x, edge_index, W_in, b_in, W1, b1, W2, b2, W3, b3, W4, b4,
           W5, b5, W6, b6, W_out, b_out):
    srcp = edge_index[0].astype(jnp.int32)
    dstp = edge_index[1].astype(jnp.int32)
    zrows = jnp.zeros((_SPW, _H), jnp.float32)
    sidx2 = 3 * srcp + (jnp.sign(srcp - dstp) + 1).astype(jnp.int32)
    base16 = jnp.concatenate(
        [x, jnp.ones((_N, 1), jnp.float32),
         jnp.zeros((_N, _H - 2), jnp.float32)], axis=1)
    e2 = jnp.zeros((3, _H), jnp.float32).at[0, 2].set(-1.0).at[2, 2].set(1.0)
    T0x = (base16[:, None, :] + e2[None, :, :]).reshape(3 * _N, _H)

    xf = jnp.repeat(jnp.pad(x[:, 0], (0, _NR - _N)).reshape(_FL, 8),
                    _H, axis=1)
    P1 = _sc_pass_first(T0x, sidx2, dstp, zrows).reshape(2, _PF, 128)
    x2f, m, ic, gm = _tc_layer1(P1, xf, W_in, b_in)

    Ws = [(W1, b1), (W2, b2), (W3, b3), (W4, b4), (W5, b5), (W6, b6)]
    for i in range(0, 6, 2):
        P = _sc_pass(x2f.reshape(_NR, _H), srcp, dstp, zrows).reshape(2, _PF, 128)
        x1f = _tc_layer(x2f, P, m, ic, gm, *_prep_w(*Ws[i]), res=None, act=True)
        P = _sc_pass(x1f.reshape(_NR, _H), srcp, dstp, zrows).reshape(2, _PF, 128)
        x2f = _tc_layer(x1f, P, m, ic, gm, *_prep_w(*Ws[i + 1]), res=x2f, act=True)

    P = _sc_pass(x2f.reshape(_NR, _H), srcp, dstp, zrows).reshape(2, _PF, 128)
    yf = _tc_layer(x2f, P, m, ic, gm, *_prep_w(W_out, b_out), res=xf,
                   act=False, compact=True)
    return yf.reshape(_NR)[:_N, None]


# speedup vs baseline: 7.2825x; 1.1501x over previous
import functools

import jax
import jax.numpy as jnp
from jax import lax
from jax.experimental import pallas as pl
from jax.experimental.pallas import tpu as pltpu
from jax.experimental.pallas import tpu_sc as plsc

_N = 50000
_H = 16
_E = 800000
_NW = 32
_EW = _E // _NW
_MC = 1000
_NMC = _EW // _MC
_NR = 50048
_SPW = _NR // 16


def _make_sc_pass(nrows):
    mesh = plsc.VectorSubcoreMesh(core_axis_name="c", subcore_axis_name="s")
    out_type = jax.ShapeDtypeStruct((2, _NR, _H), jnp.float32)
    mc, nmc = _MC, _NMC
    scratch = [
        pltpu.VMEM((2, mc), jnp.int32),
        pltpu.VMEM((2, mc), jnp.int32),
        pltpu.VMEM((2, mc, _H), jnp.float32),
        pltpu.VMEM_SHARED((_NR, _H), jnp.float32),
        pltpu.SemaphoreType.DMA((2,)),
    ]
    del nrows

    def body_fn(table, srcp, dstp, zrows, out, src_v, dst_v, rows_v, acc, sem):
        c = lax.axis_index("c")
        s = lax.axis_index("s")
        wid = s * 2 + c
        pltpu.sync_copy(zrows, acc.at[pl.ds(s * _SPW, _SPW), :])
        plsc.subcore_barrier()
        base = wid * _EW

        def fetch(j, p):
            pltpu.sync_copy(srcp.at[pl.ds(base + j * mc, mc)], src_v.at[p])
            pltpu.sync_copy(dstp.at[pl.ds(base + j * mc, mc)], dst_v.at[p])
            pltpu.async_copy(table.at[src_v.at[p]], rows_v.at[p], sem.at[p])

        def consume(p):
            pltpu.make_async_copy(
                table.at[src_v.at[p]], rows_v.at[p], sem.at[p]).wait()
            pltpu.sync_copy(rows_v.at[p], acc.at[dst_v.at[p]], add=True)

        fetch(0, 0)

        def body(t, carry):
            for p in (0, 1):
                tc = 2 * t + p

                @pl.when(tc + 1 < nmc)
                def _():
                    fetch(tc + 1, 1 - p)

                consume(p)
            return carry

        lax.fori_loop(0, nmc // 2, body, 0)
        if nmc % 2:
            consume(0)
        plsc.subcore_barrier()
        pltpu.sync_copy(acc.at[pl.ds(s * _SPW, _SPW), :],
                        out.at[c, pl.ds(s * _SPW, _SPW), :])

    return pl.kernel(
        body_fn,
        mesh=mesh,
        out_type=out_type,
        scratch_types=scratch,
        compiler_params=pltpu.CompilerParams(use_tc_tiling_on_sc=False),
    )


_sc_pass = _make_sc_pass(_N)


def _make_sc_first():
    mesh = plsc.VectorSubcoreMesh(core_axis_name="c", subcore_axis_name="s")
    mc, nmc = _MC, _NMC

    @functools.partial(
        pl.kernel,
        mesh=mesh,
        out_type=jax.ShapeDtypeStruct((2, _NR, _H), jnp.float32),
        scratch_types=[
            pltpu.VMEM((2, mc), jnp.int32),
            pltpu.VMEM((2, mc), jnp.int32),
            pltpu.VMEM((2, mc), jnp.int32),
            pltpu.VMEM((2, mc, _H), jnp.float32),
            pltpu.VMEM((2, mc, _H), jnp.float32),
            pltpu.VMEM_SHARED((_NR, _H), jnp.float32),
            pltpu.SemaphoreType.DMA((2,)),
            pltpu.SemaphoreType.DMA((2,)),
        ],
        compiler_params=pltpu.CompilerParams(use_tc_tiling_on_sc=False),
    )
    def body_fn(table, tab3k, srcp, cgidx, dstp, zrows, out,
                src_v, cg_v, dst_v, rows_v, cgrows_v, acc, sem, sem2):
        c = lax.axis_index("c")
        s = lax.axis_index("s")
        wid = s * 2 + c
        pltpu.sync_copy(zrows, acc.at[pl.ds(s * _SPW, _SPW), :])
        plsc.subcore_barrier()
        base = wid * _EW

        def fetch(j, p):
            sl = pl.ds(base + j * mc, mc)
            pltpu.sync_copy(srcp.at[sl], src_v.at[p])
            pltpu.sync_copy(cgidx.at[sl], cg_v.at[p])
            pltpu.sync_copy(dstp.at[sl], dst_v.at[p])
            pltpu.async_copy(table.at[src_v.at[p]], rows_v.at[p], sem.at[p])
            pltpu.async_copy(tab3k.at[cg_v.at[p]], cgrows_v.at[p], sem2.at[p])

        def consume(p):
            pltpu.make_async_copy(
                table.at[src_v.at[p]], rows_v.at[p], sem.at[p]).wait()
            pltpu.sync_copy(rows_v.at[p], acc.at[dst_v.at[p]], add=True)
            pltpu.make_async_copy(
                tab3k.at[cg_v.at[p]], cgrows_v.at[p], sem2.at[p]).wait()
            pltpu.sync_copy(cgrows_v.at[p], acc.at[dst_v.at[p]], add=True)

        fetch(0, 0)

        def body(t, carry):
            for p in (0, 1):
                tc = 2 * t + p

                @pl.when(tc + 1 < nmc)
                def _():
                    fetch(tc + 1, 1 - p)

                consume(p)
            return carry

        lax.fori_loop(0, nmc // 2, body, 0)
        if nmc % 2:
            consume(0)
        plsc.subcore_barrier()
        pltpu.sync_copy(acc.at[pl.ds(s * _SPW, _SPW), :],
                        out.at[c, pl.ds(s * _SPW, _SPW), :])

    return body_fn


_sc_pass_first = _make_sc_first()


_PF = _NR * _H // 128
_FL = _PF
_BR = _PF
_GRID = 1


def _l1_body(P_ref, xf_ref, wdT_ref, wbT_ref, wcT_ref, bT_ref,
             B0_ref, B1_ref, B2_ref, x2_ref, m_ref, ic_ref, gm_ref):
    Pb = P_ref[0] + P_ref[1]
    f32 = jnp.float32
    sB = jnp.dot(Pb, B0_ref[...], preferred_element_type=f32)
    cB = jnp.dot(Pb, B1_ref[...], preferred_element_type=f32)
    gB = jnp.dot(Pb, B2_ref[...], preferred_element_type=f32)
    ic = 1.0 / jnp.maximum(cB, 1.0)
    m = (cB > 0.0).astype(f32)
    gm = gB * ic
    x2_ref[...] = (m * (xf_ref[...] * wdT_ref[...] + bT_ref[...])
                   + (ic * sB) * wbT_ref[...] + gm * wcT_ref[...])
    m_ref[...] = m
    ic_ref[...] = ic
    gm_ref[...] = gm


def _tc_layer1(P, xf, W_in, b_in):
    wd = W_in[0] - W_in[1]
    ey = jnp.eye(8, dtype=jnp.float32)
    tile = lambda v: jnp.tile(v, 8)[None, :]
    Bs = []
    for k in range(3):
        Mk = jnp.zeros((_H, _H), jnp.float32).at[k].set(1.0)
        Bs.append(jnp.kron(ey, Mk))
    fspec = pl.BlockSpec((_BR, 128), lambda i: (i, 0))
    wspec = pl.BlockSpec((128, 128), lambda i: (0, 0))
    sspec = pl.BlockSpec((1, 128), lambda i: (0, 0))
    fshape = jax.ShapeDtypeStruct((_FL, 128), jnp.float32)
    return pl.pallas_call(
        _l1_body,
        grid=(_GRID,),
        in_specs=[pl.BlockSpec((2, _PF, 128), lambda i: (0, 0, 0)),
                  fspec, sspec, sspec, sspec, sspec, wspec, wspec, wspec],
        out_specs=[fspec, fspec, fspec, fspec],
        out_shape=[fshape, fshape, fshape, fshape],
    )(P, xf, tile(wd), tile(W_in[1]), tile(W_in[2]), tile(b_in), *Bs)


def _make_layer_body(act, has_res, compact):
    def body(h_ref, P_ref, m_ref, ic_ref, gm_ref, *rest):
        if has_res:
            res_ref = rest[0]
            rest = rest[1:]
        if compact:
            cm_ref = rest[0]
            rest = rest[1:]
        wd_ref, wb_ref, wc_ref, b_ref, o_ref = rest
        P = P_ref[0] + P_ref[1]
        o = (m_ref[...] * (jnp.dot(h_ref[...], wd_ref[...],
                                   preferred_element_type=jnp.float32)
                           + b_ref[...])
             + jnp.dot(ic_ref[...] * P, wb_ref[...],
                       preferred_element_type=jnp.float32)
             + gm_ref[...] * wc_ref[...])
        if has_res:
            o = o + res_ref[...]
        if act:
            o = jnp.where(o >= 0, o, 0.01 * o)
        if compact:
            o = jnp.dot(o, cm_ref[...], preferred_element_type=jnp.float32)
        o_ref[...] = o
    return body


def _tc_layer(h, P, m, ic, gm, WdBD, WbBD, wcT, bT, res, act,
              compact=False):
    fspec = pl.BlockSpec((_BR, 128), lambda i: (i, 0))
    wspec = pl.BlockSpec((128, 128), lambda i: (0, 0))
    sspec = pl.BlockSpec((1, 128), lambda i: (0, 0))
    ins = [h, P, m, ic, gm]
    specs = [fspec, pl.BlockSpec((2, _BR, 128), lambda i: (0, i, 0)),
             fspec, fspec, fspec]
    if res is not None:
        ins.append(res)
        specs.append(fspec)
    if compact:
        cmat = jnp.kron(jnp.eye(8, dtype=jnp.float32),
                        jnp.zeros((_H, 1), jnp.float32).at[0, 0].set(1.0))
        ins.append(cmat)
        specs.append(pl.BlockSpec((128, 8), lambda i: (0, 0)))
        out_spec = pl.BlockSpec((_BR, 8), lambda i: (i, 0))
        out_shape = jax.ShapeDtypeStruct((_FL, 8), jnp.float32)
    else:
        out_spec = fspec
        out_shape = jax.ShapeDtypeStruct((_FL, 128), jnp.float32)
    ins += [WdBD, WbBD, wcT, bT]
    specs += [wspec, wspec, sspec, sspec]
    return pl.pallas_call(
        _make_layer_body(act, res is not None, compact),
        grid=(_GRID,),
        in_specs=specs,
        out_specs=out_spec,
        out_shape=out_shape,
    )(*ins)


def _prep_w(W, b):
    ey = jnp.eye(8, dtype=jnp.float32)
    wd = W[:_H] - W[_H:2 * _H]
    wb = W[_H:2 * _H]
    if W.shape[1] == 1:
        on = jnp.ones((1, _H), jnp.float32)
        wd, wb = wd @ on, wb @ on
        wc = jnp.tile(W[2 * _H] @ on, 8)[None, :]
        bT = jnp.tile(b @ on, 8).reshape(1, 128)
    else:
        wc = jnp.tile(W[2 * _H], 8)[None, :]
        bT = jnp.tile(b, 8)[None, :]
    return jnp.kron(ey, wd), jnp.kron(ey, wb), wc, bT


def kernel(x, edge_index, W_in, b_in, W1, b1, W2, b2, W3, b3, W4, b4,
           W5, b5, W6, b6, W_out, b_out):
    srcp = edge_index[0].astype(jnp.int32)
    dstp = edge_index[1].astype(jnp.int32)
    zrows = jnp.zeros((_SPW, _H), jnp.float32)
    cgidx = (3 * (jnp.arange(_E, dtype=jnp.int32) & 1023)
             + jnp.sign(srcp - dstp) + 1)
    tab3k = jnp.tile(
        jnp.zeros((3, _H), jnp.float32)
        .at[0, 2].set(-1.0).at[2, 2].set(1.0), (1024, 1))
    T0 = jnp.concatenate(
        [x, jnp.ones((_N, 1), jnp.float32),
         jnp.zeros((_N, _H - 2), jnp.float32)], axis=1)

    xf = jnp.repeat(jnp.pad(x[:, 0], (0, _NR - _N)).reshape(_FL, 8),
                    _H, axis=1)
    P1 = _sc_pass_first(T0, tab3k, srcp, cgidx, dstp,
                        zrows).reshape(2, _PF, 128)
    x2f, m, ic, gm = _tc_layer1(P1, xf, W_in, b_in)

    Ws = [(W1, b1), (W2, b2), (W3, b3), (W4, b4), (W5, b5), (W6, b6)]
    for i in range(0, 6, 2):
        P = _sc_pass(x2f.reshape(_NR, _H), srcp, dstp, zrows).reshape(2, _PF, 128)
        x1f = _tc_layer(x2f, P, m, ic, gm, *_prep_w(*Ws[i]), res=None, act=True)
        P = _sc_pass(x1f.reshape(_NR, _H), srcp, dstp, zrows).reshape(2, _PF, 128)
        x2f = _tc_layer(x1f, P, m, ic, gm, *_prep_w(*Ws[i + 1]), res=x2f, act=True)

    P = _sc_pass(x2f.reshape(_NR, _H), srcp, dstp, zrows).reshape(2, _PF, 128)
    yf = _tc_layer(x2f, P, m, ic, gm, *_prep_w(W_out, b_out), res=xf,
                   act=False, compact=True)
    return yf.reshape(_NR)[:_N, None]


# speedup vs baseline: 7.7304x; 1.0615x over previous
import functools

import jax
import jax.numpy as jnp
from jax import lax
from jax.experimental import pallas as pl
from jax.experimental.pallas import tpu as pltpu
from jax.experimental.pallas import tpu_sc as plsc

_N = 50000
_H = 16
_E = 800000
_NW = 32
_EW = _E // _NW
_MC = 1000
_NMC = _EW // _MC
_NR = 50048
_SPW = _NR // 16


def _make_sc_pass(nrows):
    mesh = plsc.VectorSubcoreMesh(core_axis_name="c", subcore_axis_name="s")
    out_type = jax.ShapeDtypeStruct((2, _NR, _H), jnp.float32)
    mc, nmc = _MC, _NMC
    scratch = [
        pltpu.VMEM((2, 2, mc), jnp.int32),
        pltpu.VMEM((2, mc, _H), jnp.float32),
        pltpu.VMEM_SHARED((_NR, _H), jnp.float32),
        pltpu.SemaphoreType.DMA((2,)),
    ]
    del nrows

    def body_fn(table, sd2, zrows, out, sd_v, rows_v, acc, sem):
        c = lax.axis_index("c")
        s = lax.axis_index("s")
        wid = s * 2 + c
        pltpu.sync_copy(zrows, acc.at[pl.ds(s * _SPW, _SPW), :])
        plsc.subcore_barrier()
        wrow = wid * nmc

        def fetch(j, p):
            pltpu.sync_copy(sd2.at[wrow + j], sd_v.at[p])
            pltpu.async_copy(table.at[sd_v.at[p, 0]], rows_v.at[p],
                             sem.at[p])

        def consume(p):
            pltpu.make_async_copy(
                table.at[sd_v.at[p, 0]], rows_v.at[p], sem.at[p]).wait()
            pltpu.sync_copy(rows_v.at[p], acc.at[sd_v.at[p, 1]], add=True)

        fetch(0, 0)

        def body(t, carry):
            for p in (0, 1):
                tc = 2 * t + p

                @pl.when(tc + 1 < nmc)
                def _():
                    fetch(tc + 1, 1 - p)

                consume(p)
            return carry

        lax.fori_loop(0, nmc // 2, body, 0)
        if nmc % 2:
            consume(0)
        plsc.subcore_barrier()
        pltpu.sync_copy(acc.at[pl.ds(s * _SPW, _SPW), :],
                        out.at[c, pl.ds(s * _SPW, _SPW), :])

    return pl.kernel(
        body_fn,
        mesh=mesh,
        out_type=out_type,
        scratch_types=scratch,
        compiler_params=pltpu.CompilerParams(use_tc_tiling_on_sc=False),
    )


_sc_pass = _make_sc_pass(_N)


def _make_sc_first():
    mesh = plsc.VectorSubcoreMesh(core_axis_name="c", subcore_axis_name="s")
    mc, nmc = _MC, _NMC

    @functools.partial(
        pl.kernel,
        mesh=mesh,
        out_type=jax.ShapeDtypeStruct((2, _NR, _H), jnp.float32),
        scratch_types=[
            pltpu.VMEM((2, mc), jnp.int32),
            pltpu.VMEM((2, mc), jnp.int32),
            pltpu.VMEM((2, mc), jnp.int32),
            pltpu.VMEM((2, mc, _H), jnp.float32),
            pltpu.VMEM((2, mc, _H), jnp.float32),
            pltpu.VMEM_SHARED((_NR, _H), jnp.float32),
            pltpu.SemaphoreType.DMA((2,)),
            pltpu.SemaphoreType.DMA((2,)),
        ],
        compiler_params=pltpu.CompilerParams(use_tc_tiling_on_sc=False),
    )
    def body_fn(table, tab3k, srcp, cgidx, dstp, zrows, out,
                src_v, cg_v, dst_v, rows_v, cgrows_v, acc, sem, sem2):
        c = lax.axis_index("c")
        s = lax.axis_index("s")
        wid = s * 2 + c
        pltpu.sync_copy(zrows, acc.at[pl.ds(s * _SPW, _SPW), :])
        plsc.subcore_barrier()
        base = wid * _EW

        def fetch(j, p):
            sl = pl.ds(base + j * mc, mc)
            pltpu.sync_copy(srcp.at[sl], src_v.at[p])
            pltpu.sync_copy(cgidx.at[sl], cg_v.at[p])
            pltpu.sync_copy(dstp.at[sl], dst_v.at[p])
            pltpu.async_copy(table.at[src_v.at[p]], rows_v.at[p], sem.at[p])
            pltpu.async_copy(tab3k.at[cg_v.at[p]], cgrows_v.at[p], sem2.at[p])

        def consume(p):
            pltpu.make_async_copy(
                table.at[src_v.at[p]], rows_v.at[p], sem.at[p]).wait()
            pltpu.sync_copy(rows_v.at[p], acc.at[dst_v.at[p]], add=True)
            pltpu.make_async_copy(
                tab3k.at[cg_v.at[p]], cgrows_v.at[p], sem2.at[p]).wait()
            pltpu.sync_copy(cgrows_v.at[p], acc.at[dst_v.at[p]], add=True)

        fetch(0, 0)

        def body(t, carry):
            for p in (0, 1):
                tc = 2 * t + p

                @pl.when(tc + 1 < nmc)
                def _():
                    fetch(tc + 1, 1 - p)

                consume(p)
            return carry

        lax.fori_loop(0, nmc // 2, body, 0)
        if nmc % 2:
            consume(0)
        plsc.subcore_barrier()
        pltpu.sync_copy(acc.at[pl.ds(s * _SPW, _SPW), :],
                        out.at[c, pl.ds(s * _SPW, _SPW), :])

    return body_fn


_sc_pass_first = _make_sc_first()


_PF = _NR * _H // 128
_FL = _PF
_BR = _PF
_GRID = 1


def _l1_body(P_ref, xf_ref, wdT_ref, wbT_ref, wcT_ref, bT_ref,
             B0_ref, B1_ref, B2_ref, x2_ref, m_ref, ic_ref, gm_ref):
    Pb = P_ref[0] + P_ref[1]
    f32 = jnp.float32
    sB = jnp.dot(Pb, B0_ref[...], preferred_element_type=f32)
    cB = jnp.dot(Pb, B1_ref[...], preferred_element_type=f32)
    gB = jnp.dot(Pb, B2_ref[...], preferred_element_type=f32)
    ic = 1.0 / jnp.maximum(cB, 1.0)
    m = (cB > 0.0).astype(f32)
    gm = gB * ic
    x2_ref[...] = (m * (xf_ref[...] * wdT_ref[...] + bT_ref[...])
                   + (ic * sB) * wbT_ref[...] + gm * wcT_ref[...])
    m_ref[...] = m
    ic_ref[...] = ic
    gm_ref[...] = gm


def _tc_layer1(P, xf, W_in, b_in):
    wd = W_in[0] - W_in[1]
    ey = jnp.eye(8, dtype=jnp.float32)
    tile = lambda v: jnp.tile(v, 8)[None, :]
    Bs = []
    for k in range(3):
        Mk = jnp.zeros((_H, _H), jnp.float32).at[k].set(1.0)
        Bs.append(jnp.kron(ey, Mk))
    fspec = pl.BlockSpec((_BR, 128), lambda i: (i, 0))
    wspec = pl.BlockSpec((128, 128), lambda i: (0, 0))
    sspec = pl.BlockSpec((1, 128), lambda i: (0, 0))
    fshape = jax.ShapeDtypeStruct((_FL, 128), jnp.float32)
    return pl.pallas_call(
        _l1_body,
        grid=(_GRID,),
        in_specs=[pl.BlockSpec((2, _PF, 128), lambda i: (0, 0, 0)),
                  fspec, sspec, sspec, sspec, sspec, wspec, wspec, wspec],
        out_specs=[fspec, fspec, fspec, fspec],
        out_shape=[fshape, fshape, fshape, fshape],
    )(P, xf, tile(wd), tile(W_in[1]), tile(W_in[2]), tile(b_in), *Bs)


def _make_layer_body(act, has_res, compact):
    def body(h_ref, P_ref, m_ref, ic_ref, gm_ref, *rest):
        if has_res:
            res_ref = rest[0]
            rest = rest[1:]
        if compact:
            cm_ref = rest[0]
            rest = rest[1:]
        wd_ref, wb_ref, wc_ref, b_ref, o_ref = rest
        P = P_ref[0] + P_ref[1]
        o = (m_ref[...] * (jnp.dot(h_ref[...], wd_ref[...],
                                   preferred_element_type=jnp.float32)
                           + b_ref[...])
             + jnp.dot(ic_ref[...] * P, wb_ref[...],
                       preferred_element_type=jnp.float32)
             + gm_ref[...] * wc_ref[...])
        if has_res:
            o = o + res_ref[...]
        if act:
            o = jnp.where(o >= 0, o, 0.01 * o)
        if compact:
            o = jnp.dot(o, cm_ref[...], preferred_element_type=jnp.float32)
        o_ref[...] = o
    return body


def _tc_layer(h, P, m, ic, gm, WdBD, WbBD, wcT, bT, res, act,
              compact=False):
    fspec = pl.BlockSpec((_BR, 128), lambda i: (i, 0))
    wspec = pl.BlockSpec((128, 128), lambda i: (0, 0))
    sspec = pl.BlockSpec((1, 128), lambda i: (0, 0))
    ins = [h, P, m, ic, gm]
    specs = [fspec, pl.BlockSpec((2, _BR, 128), lambda i: (0, i, 0)),
             fspec, fspec, fspec]
    if res is not None:
        ins.append(res)
        specs.append(fspec)
    if compact:
        cmat = jnp.kron(jnp.eye(8, dtype=jnp.float32),
                        jnp.zeros((_H, 1), jnp.float32).at[0, 0].set(1.0))
        ins.append(cmat)
        specs.append(pl.BlockSpec((128, 8), lambda i: (0, 0)))
        out_spec = pl.BlockSpec((_BR, 8), lambda i: (i, 0))
        out_shape = jax.ShapeDtypeStruct((_FL, 8), jnp.float32)
    else:
        out_spec = fspec
        out_shape = jax.ShapeDtypeStruct((_FL, 128), jnp.float32)
    ins += [WdBD, WbBD, wcT, bT]
    specs += [wspec, wspec, sspec, sspec]
    return pl.pallas_call(
        _make_layer_body(act, res is not None, compact),
        grid=(_GRID,),
        in_specs=specs,
        out_specs=out_spec,
        out_shape=out_shape,
    )(*ins)


def _prep_w(W, b):
    ey = jnp.eye(8, dtype=jnp.float32)
    wd = W[:_H] - W[_H:2 * _H]
    wb = W[_H:2 * _H]
    if W.shape[1] == 1:
        on = jnp.ones((1, _H), jnp.float32)
        wd, wb = wd @ on, wb @ on
        wc = jnp.tile(W[2 * _H] @ on, 8)[None, :]
        bT = jnp.tile(b @ on, 8).reshape(1, 128)
    else:
        wc = jnp.tile(W[2 * _H], 8)[None, :]
        bT = jnp.tile(b, 8)[None, :]
    return jnp.kron(ey, wd), jnp.kron(ey, wb), wc, bT


def kernel(x, edge_index, W_in, b_in, W1, b1, W2, b2, W3, b3, W4, b4,
           W5, b5, W6, b6, W_out, b_out):
    srcp = edge_index[0].astype(jnp.int32)
    dstp = edge_index[1].astype(jnp.int32)
    zrows = jnp.zeros((_SPW, _H), jnp.float32)
    cgidx = (3 * (jnp.arange(_E, dtype=jnp.int32) & 1023)
             + jnp.sign(srcp - dstp) + 1)
    tab3k = jnp.tile(
        jnp.zeros((3, _H), jnp.float32)
        .at[0, 2].set(-1.0).at[2, 2].set(1.0), (1024, 1))
    T0 = jnp.concatenate(
        [x, jnp.ones((_N, 1), jnp.float32),
         jnp.zeros((_N, _H - 2), jnp.float32)], axis=1)

    xf = jnp.repeat(jnp.pad(x[:, 0], (0, _NR - _N)).reshape(_FL, 8),
                    _H, axis=1)
    sd2 = jnp.stack([srcp, dstp]).reshape(2, _E // _MC, _MC).transpose(1, 0, 2)
    P1 = _sc_pass_first(T0, tab3k, srcp, cgidx, dstp,
                        zrows).reshape(2, _PF, 128)
    x2f, m, ic, gm = _tc_layer1(P1, xf, W_in, b_in)

    Ws = [(W1, b1), (W2, b2), (W3, b3), (W4, b4), (W5, b5), (W6, b6)]
    for i in range(0, 6, 2):
        P = _sc_pass(x2f.reshape(_NR, _H), sd2, zrows).reshape(2, _PF, 128)
        x1f = _tc_layer(x2f, P, m, ic, gm, *_prep_w(*Ws[i]), res=None, act=True)
        P = _sc_pass(x1f.reshape(_NR, _H), sd2, zrows).reshape(2, _PF, 128)
        x2f = _tc_layer(x1f, P, m, ic, gm, *_prep_w(*Ws[i + 1]), res=x2f, act=True)

    P = _sc_pass(x2f.reshape(_NR, _H), sd2, zrows).reshape(2, _PF, 128)
    yf = _tc_layer(x2f, P, m, ic, gm, *_prep_w(W_out, b_out), res=xf,
                   act=False, compact=True)
    return yf.reshape(_NR)[:_N, None]
